# baseline (speedup 1.0000x reference)
import jax
import jax.numpy as jnp
from jax import lax
from jax.experimental import pallas as pl
from jax.experimental.pallas import tpu as pltpu, tpu_sc as plsc

MAX_LEN = 512
EMBED = 64
BATCH = 1024
SEQ = 512

N = BATCH * SEQ
NC, NS = 2, 16
NW = NC * NS
ROWS_PER_W = N // NW
CHUNK = 128
CHUNKS = ROWS_PER_W // CHUNK
LANES = 16
NBUF = 4


def _add_pos(row_ref, pos_ref, pos_row0):
    @pl.loop(0, CHUNK, unroll=8)
    def _(r):
        pr = pos_row0 + r
        for c in range(EMBED // LANES):
            sl = pl.ds(c * LANES, LANES)
            row_ref[r, sl] = row_ref[r, sl] + pos_ref[pr, sl]


def _body(x_hbm, tok_hbm, pos_hbm, out_hbm, pos_v, idx_v, rows, gsems, osems):
    cid = lax.axis_index("c")
    sid = lax.axis_index("s")
    wid = sid * NC + cid
    base_w = wid * ROWS_PER_W

    pltpu.sync_copy(pos_hbm, pos_v)
    pltpu.sync_copy(x_hbm.at[pl.ds(wid * CHUNKS, CHUNKS)], idx_v)

    def issue_gather(c, b):
        pltpu.sync_copy(pos_hbm.at[pl.ds(b * CHUNK, CHUNK)], rows[b])
        pltpu.async_copy(tok_hbm.at[idx_v.at[c]], rows[b], gsems[b], add=True)

    def wait_gather(b):
        pltpu.make_async_copy(tok_hbm.at[idx_v.at[0]], rows[b], gsems[b]).wait()

    def issue_scatter(c, b):
        pltpu.async_copy(rows[b], out_hbm.at[pl.ds(base_w + c * CHUNK, CHUNK)],
                         osems[b])

    def wait_scatter(b):
        pltpu.make_async_copy(rows[b], out_hbm.at[pl.ds(base_w, CHUNK)],
                              osems[b]).wait()

    def finish_chunk(c, b):
        wait_gather(b)
        issue_scatter(c, b)

    for b in range(NBUF):
        issue_gather(b, b)
    finish_chunk(0, 0)
    finish_chunk(1, 1)

    @pl.loop(NBUF, CHUNKS, step=NBUF)
    def _(go):
        for b in range(NBUF):
            c = go + b
            wait_scatter(b)
            issue_gather(c, b)
            b2 = (b + NBUF - 2) % NBUF
            finish_chunk(c - 2, b2)

    finish_chunk(CHUNKS - 2, (CHUNKS - 2) % NBUF)
    finish_chunk(CHUNKS - 1, (CHUNKS - 1) % NBUF)
    for b in range(NBUF):
        wait_scatter(b)


def kernel(x, token_table, pos_table):
    xf = x.reshape(NW * CHUNKS, CHUNK)
    mesh = plsc.VectorSubcoreMesh(
        core_axis_name="c", subcore_axis_name="s", num_cores=NC, num_subcores=NS
    )

    def body(x_ref, tok_ref, pos_ref, out_ref, pos_v, idx_v,
             r0, r1, r2, r3, g0, g1, g2, g3, o0, o1, o2, o3):
        _body(x_ref, tok_ref, pos_ref, out_ref, pos_v, idx_v,
              [r0, r1, r2, r3], [g0, g1, g2, g3], [o0, o1, o2, o3])

    run = pl.kernel(
        body,
        out_type=jax.ShapeDtypeStruct((N, EMBED), jnp.float32),
        mesh=mesh,
        scratch_types=[
            pltpu.VMEM((MAX_LEN, EMBED), jnp.float32),
            pltpu.VMEM((CHUNKS, CHUNK), jnp.int32),
        ] + [pltpu.VMEM((CHUNK, EMBED), jnp.float32) for _ in range(NBUF)]
          + [pltpu.SemaphoreType.DMA for _ in range(2 * NBUF)],
        compiler_params=pltpu.CompilerParams(use_tc_tiling_on_sc=False),
    )
    out = run(xf, token_table, pos_table)
    return out.reshape(BATCH, SEQ, EMBED)


# speedup vs baseline: 1.3762x; 1.3762x over previous
import jax
import jax.numpy as jnp
from jax import lax
from jax.experimental import pallas as pl
from jax.experimental.pallas import tpu as pltpu, tpu_sc as plsc

MAX_LEN = 512
EMBED = 64
BATCH = 1024
SEQ = 512

N = BATCH * SEQ
NC, NS = 2, 16
NW = NC * NS
ROWS_PER_W = N // NW
CHUNK = 128
CHUNKS = ROWS_PER_W // CHUNK
POSW = MAX_LEN // CHUNK
NBUF = 8


def _body(x_hbm, tok_hbm, pos_hbm, out_hbm, pos_sh, idx_v, rows,
          isems, gsems, osems):
    cid = lax.axis_index("c")
    sid = lax.axis_index("s")
    wid = sid * NC + cid
    base_w = wid * ROWS_PER_W

    @pl.when(sid == 0)
    def _():
        pltpu.sync_copy(pos_hbm, pos_sh)

    plsc.subcore_barrier()

    pltpu.sync_copy(x_hbm.at[pl.ds(wid * CHUNKS, CHUNKS)], idx_v)

    def pos_src(b):
        return pos_sh.at[pl.ds((b % POSW) * CHUNK, CHUNK)]

    def issue_init(b):
        pltpu.async_copy(pos_src(b), rows[b], isems[b])

    def issue_gather(c, b):
        pltpu.make_async_copy(pos_src(b), rows[b], isems[b]).wait()
        pltpu.async_copy(tok_hbm.at[idx_v.at[c]], rows[b], gsems[b], add=True)

    def issue_scatter(c, b):
        pltpu.make_async_copy(tok_hbm.at[idx_v.at[0]], rows[b], gsems[b]).wait()
        pltpu.async_copy(rows[b], out_hbm.at[pl.ds(base_w + c * CHUNK, CHUNK)],
                         osems[b])

    def wait_scatter(b):
        pltpu.make_async_copy(rows[b], out_hbm.at[pl.ds(base_w, CHUNK)],
                              osems[b]).wait()

    for c in range(NBUF):
        issue_init(c)
        if c >= 2:
            issue_gather(c - 2, c - 2)
        if c >= 4:
            issue_scatter(c - 4, c - 4)

    @pl.loop(NBUF, CHUNKS, step=NBUF)
    def _(go):
        for b in range(NBUF):
            c = go + b
            wait_scatter(b)
            issue_init(b)
            b2 = (b + NBUF - 2) % NBUF
            issue_gather(c - 2, b2)
            b4 = (b + NBUF - 4) % NBUF
            issue_scatter(c - 4, b4)

    for c in range(CHUNKS, CHUNKS + 4):
        if c - 2 < CHUNKS:
            issue_gather(c - 2, (c - 2) % NBUF)
        issue_scatter(c - 4, (c - 4) % NBUF)
    for b in range(NBUF):
        wait_scatter(b)


def kernel(x, token_table, pos_table):
    xf = x.reshape(NW * CHUNKS, CHUNK)
    mesh = plsc.VectorSubcoreMesh(
        core_axis_name="c", subcore_axis_name="s", num_cores=NC, num_subcores=NS
    )

    def body(x_ref, tok_ref, pos_ref, out_ref, pos_sh, idx_v, *rest):
        rows = list(rest[:NBUF])
        isems = list(rest[NBUF:2 * NBUF])
        gsems = list(rest[2 * NBUF:3 * NBUF])
        osems = list(rest[3 * NBUF:4 * NBUF])
        _body(x_ref, tok_ref, pos_ref, out_ref, pos_sh, idx_v, rows,
              isems, gsems, osems)

    run = pl.kernel(
        body,
        out_type=jax.ShapeDtypeStruct((N, EMBED), jnp.float32),
        mesh=mesh,
        scratch_types=[
            pltpu.VMEM_SHARED((MAX_LEN, EMBED), jnp.float32),
            pltpu.VMEM((CHUNKS, CHUNK), jnp.int32),
        ] + [pltpu.VMEM((CHUNK, EMBED), jnp.float32) for _ in range(NBUF)]
          + [pltpu.SemaphoreType.DMA for _ in range(3 * NBUF)],
        compiler_params=pltpu.CompilerParams(use_tc_tiling_on_sc=False),
    )
    out = run(xf, token_table, pos_table)
    return out.reshape(BATCH, SEQ, EMBED)
